# baseline (speedup 1.0000x reference)
import jax
import jax.numpy as jnp
from jax import lax
from jax.experimental import pallas as pl
from jax.experimental.pallas import tpu as pltpu

H = W = 50
NPIX = H * W
CIN = 512
GUARD = 56
NG = GUARD * 2 + NPIX
CREG = 36
CCLS = 18
CHEAD = CREG + CCLS

_CT_CHAN = (((1,), (1,)), ((), ()))


def _rpn_body(x_ref, w_ref, bsw_ref, whead_ref, bhead_ref,
              regc_ref, clsc_ref, xg_ref, acc_ref):
    k = pl.program_id(0)

    @pl.when(k == 0)
    def _init():
        xg_ref[0:GUARD, :] = jnp.zeros((GUARD, CIN), jnp.bfloat16)
        xg_ref[GUARD:GUARD + NPIX, :] = x_ref[...].astype(jnp.bfloat16)
        xg_ref[GUARD + NPIX:NG, :] = jnp.zeros((GUARD, CIN), jnp.bfloat16)
        acc_ref[...] = jnp.zeros((NPIX, CIN), jnp.float32)

    dx = k % 3
    xs = lax.switch(
        k,
        [lambda s=GUARD + (t // 3 - 1) * W + (t % 3 - 1):
             xg_ref[s:s + NPIX, :]
         for t in range(9)])
    y = lax.dot_general(xs, w_ref[0], _CT_CHAN,
                        preferred_element_type=jnp.float32)

    pmod = lax.broadcasted_iota(jnp.int32, (NPIX, 1), 0) % W
    edge = jnp.where(dx == 0, 0, W - 1)
    mask = jnp.logical_or(dx == 1, pmod != edge)
    acc_ref[...] += jnp.where(mask, y, 0.0)

    @pl.when(k == 8)
    def _head():
        feat = jnp.maximum(acc_ref[...] + bsw_ref[...], 0.0)
        feat = feat.astype(jnp.bfloat16)
        outc = lax.dot_general(whead_ref[...].astype(jnp.bfloat16), feat,
                               _CT_CHAN,
                               preferred_element_type=jnp.float32)
        outc += bhead_ref[...]
        regc_ref[...] = outc[0:CREG]
        clsc_ref[...] = outc[CREG:CHEAD]


def kernel(x, W_sw, b_sw, W_cls, b_cls, W_reg, b_reg):
    x2d = jnp.transpose(x[0], (1, 2, 0)).reshape(NPIX, CIN)
    x2d = x2d.astype(jnp.bfloat16)
    w9 = jnp.transpose(W_sw, (2, 3, 0, 1)).reshape(9, CIN, CIN)
    whead = jnp.concatenate(
        [W_reg[:, :, 0, 0], W_cls[:, :, 0, 0]], axis=0)
    bsw = b_sw.reshape(1, CIN)
    bhead = jnp.concatenate([b_reg, b_cls]).reshape(CHEAD, 1)

    regc, clsc = pl.pallas_call(
        _rpn_body,
        grid=(9,),
        out_shape=(jax.ShapeDtypeStruct((CREG, NPIX), jnp.float32),
                   jax.ShapeDtypeStruct((CCLS, NPIX), jnp.float32)),
        in_specs=[
            pl.BlockSpec((NPIX, CIN), lambda k: (0, 0)),
            pl.BlockSpec((1, CIN, CIN), lambda k: (k, 0, 0)),
            pl.BlockSpec((1, CIN), lambda k: (0, 0)),
            pl.BlockSpec((CHEAD, CIN), lambda k: (0, 0)),
            pl.BlockSpec((CHEAD, 1), lambda k: (0, 0)),
        ],
        out_specs=(pl.BlockSpec((CREG, NPIX), lambda k: (0, 0)),
                   pl.BlockSpec((CCLS, NPIX), lambda k: (0, 0))),
        scratch_shapes=[
            pltpu.VMEM((NG, CIN), jnp.bfloat16),
            pltpu.VMEM((NPIX, CIN), jnp.float32),
        ],
    )(x2d, w9, bsw, whead, bhead)

    reg = jnp.transpose(regc.reshape(9, 4, NPIX), (2, 0, 1)).reshape(
        1, NPIX * 9, 4)
    cls = jnp.transpose(clsc.reshape(9, 2, NPIX), (2, 0, 1)).reshape(
        1, NPIX * 9, 2)
    return (reg, cls)


# speedup vs baseline: 1.7571x; 1.7571x over previous
import jax
import jax.numpy as jnp
from jax import lax
from jax.experimental import pallas as pl
from jax.experimental.pallas import tpu as pltpu

H = W = 50
NPIX = H * W
CIN = 512
GUARD = 56
CREG = 36
CCLS = 18
CHEAD = CREG + CCLS

_CT_CHAN = (((1,), (1,)), ((), ()))


def _rpn_body(x_ref, w9_ref, bsw_ref, whead_ref, bhead_ref,
              regc_ref, clsc_ref):
    xb = x_ref[...]
    zg = jnp.zeros((GUARD, CIN), jnp.bfloat16)
    xg = jnp.concatenate([zg, xb, zg], axis=0)
    w9 = w9_ref[...]

    pmod = lax.broadcasted_iota(jnp.int32, (NPIX, 1), 0) % W
    m_left = pmod != 0
    m_right = pmod != W - 1

    acc = jnp.zeros((NPIX, CIN), jnp.float32)
    for dy in range(3):
        for dx in range(3):
            start = GUARD + (dy - 1) * W + (dx - 1)
            y = lax.dot_general(xg[start:start + NPIX], w9[3 * dy + dx],
                                _CT_CHAN,
                                preferred_element_type=jnp.float32)
            if dx == 0:
                y = jnp.where(m_left, y, 0.0)
            elif dx == 2:
                y = jnp.where(m_right, y, 0.0)
            acc += y

    feat = jnp.maximum(acc + bsw_ref[...], 0.0).astype(jnp.bfloat16)
    outc = lax.dot_general(whead_ref[...].astype(jnp.bfloat16), feat,
                           _CT_CHAN,
                           preferred_element_type=jnp.float32)
    outc += bhead_ref[...]
    regc_ref[...] = outc[0:CREG]
    clsc_ref[...] = outc[CREG:CHEAD]


def kernel(x, W_sw, b_sw, W_cls, b_cls, W_reg, b_reg):
    x2d = jnp.transpose(x[0], (1, 2, 0)).reshape(NPIX, CIN)
    x2d = x2d.astype(jnp.bfloat16)
    w9 = jnp.transpose(W_sw, (2, 3, 0, 1)).reshape(9, CIN, CIN)
    w9 = w9.astype(jnp.bfloat16)
    whead = jnp.concatenate(
        [W_reg[:, :, 0, 0], W_cls[:, :, 0, 0]], axis=0)
    bsw = b_sw.reshape(1, CIN)
    bhead = jnp.concatenate([b_reg, b_cls]).reshape(CHEAD, 1)

    regc, clsc = pl.pallas_call(
        _rpn_body,
        out_shape=(jax.ShapeDtypeStruct((CREG, NPIX), jnp.float32),
                   jax.ShapeDtypeStruct((CCLS, NPIX), jnp.float32)),
        in_specs=[pl.BlockSpec(memory_space=pltpu.VMEM)] * 5,
        out_specs=(pl.BlockSpec(memory_space=pltpu.VMEM),
                   pl.BlockSpec(memory_space=pltpu.VMEM)),
    )(x2d, w9, bsw, whead, bhead)

    reg = jnp.transpose(regc.reshape(9, 4, NPIX), (2, 0, 1)).reshape(
        1, NPIX * 9, 4)
    cls = jnp.transpose(clsc.reshape(9, 2, NPIX), (2, 0, 1)).reshape(
        1, NPIX * 9, 2)
    return (reg, cls)


# speedup vs baseline: 1.8068x; 1.0283x over previous
import jax
import jax.numpy as jnp
from jax import lax
from jax.experimental import pallas as pl
from jax.experimental.pallas import tpu as pltpu

H = W = 50
NPIX = H * W
CIN = 512
GUARD = 56
CREG = 36
CCLS = 18
CHEAD = CREG + CCLS

_CT_CHAN = (((1,), (1,)), ((), ()))


def _rpn_body(x_ref, w9_ref, bsw_ref, whead_ref, bhead_ref,
              regc_ref, clsc_ref):
    xb = x_ref[...].astype(jnp.bfloat16)
    zg = jnp.zeros((GUARD, CIN), jnp.bfloat16)
    xg = jnp.concatenate([zg, xb, zg], axis=0)
    w9 = w9_ref[...].astype(jnp.bfloat16)

    pmod = lax.broadcasted_iota(jnp.int32, (NPIX, 1), 0) % W
    m_left = pmod != 0
    m_right = pmod != W - 1

    acc = jnp.zeros((NPIX, CIN), jnp.float32)
    for dy in range(3):
        for dx in range(3):
            start = GUARD + (dy - 1) * W + (dx - 1)
            y = lax.dot_general(xg[start:start + NPIX], w9[3 * dy + dx],
                                _CT_CHAN,
                                preferred_element_type=jnp.float32)
            if dx == 0:
                y = jnp.where(m_left, y, 0.0)
            elif dx == 2:
                y = jnp.where(m_right, y, 0.0)
            acc += y

    feat = jnp.maximum(acc + bsw_ref[...], 0.0).astype(jnp.bfloat16)
    outc = lax.dot_general(whead_ref[...].astype(jnp.bfloat16), feat,
                           _CT_CHAN,
                           preferred_element_type=jnp.float32)
    outc += bhead_ref[...]
    regc_ref[...] = outc[0:CREG]
    clsc_ref[...] = outc[CREG:CHEAD]


def kernel(x, W_sw, b_sw, W_cls, b_cls, W_reg, b_reg):
    x2d = jnp.transpose(x[0], (1, 2, 0)).reshape(NPIX, CIN)
    w9 = jnp.transpose(W_sw, (2, 3, 0, 1)).reshape(9, CIN, CIN)
    whead = jnp.concatenate(
        [W_reg[:, :, 0, 0], W_cls[:, :, 0, 0]], axis=0)
    bsw = b_sw.reshape(1, CIN)
    bhead = jnp.concatenate([b_reg, b_cls]).reshape(CHEAD, 1)

    regc, clsc = pl.pallas_call(
        _rpn_body,
        out_shape=(jax.ShapeDtypeStruct((CREG, NPIX), jnp.float32),
                   jax.ShapeDtypeStruct((CCLS, NPIX), jnp.float32)),
        in_specs=[pl.BlockSpec(memory_space=pltpu.VMEM)] * 5,
        out_specs=(pl.BlockSpec(memory_space=pltpu.VMEM),
                   pl.BlockSpec(memory_space=pltpu.VMEM)),
    )(x2d, w9, bsw, whead, bhead)

    reg = jnp.transpose(regc.reshape(9, 4, NPIX), (2, 0, 1)).reshape(
        1, NPIX * 9, 4)
    cls = jnp.transpose(clsc.reshape(9, 2, NPIX), (2, 0, 1)).reshape(
        1, NPIX * 9, 2)
    return (reg, cls)
